# Initial kernel scaffold; baseline (speedup 1.0000x reference)
import jax
import jax.numpy as jnp
from jax.experimental import pallas as pl


def kernel(x):
    raise NotImplementedError("write your pallas kernel here")


# speedup vs baseline: 8.1825x; 8.1825x over previous
import functools

import jax
import jax.numpy as jnp
from jax import lax
from jax.experimental import pallas as pl
from jax.experimental.pallas import tpu as pltpu
from jax.experimental.pallas import tpu_sc as plsc

NC = 2
NS = 16
L = 16
NW = NC * NS
N = 128 * 8192
SHARD = N // NW
ITERS = SHARD // L
KK = 8192

_MESH = plsc.VectorSubcoreMesh(
    core_axis_name="c", subcore_axis_name="s", num_cores=NC, num_subcores=NS)
_SC_PARAMS = pltpu.CompilerParams(needs_layout_passes=False)


def _wid():
    return lax.axis_index("s") * NC + lax.axis_index("c")


def _bits_of(xv):
    return jnp.where(xv > 0.0, plsc.bitcast(xv, jnp.int32), 0)


def _zero_hist(hist, nbins):
    z = jnp.zeros((L,), jnp.int32)

    def zbody(i, c):
        hist[pl.ds(i * L, L)] = z
        return c

    lax.fori_loop(0, nbins // L, zbody, 0, unroll=4)


def _hist_body(x_hbm, sel_hbm, out_hbm, xs, hist, selv, *, nbins, level):
    wid = _wid()
    pltpu.sync_copy(x_hbm.at[pl.ds(wid * SHARD, SHARD)], xs)
    _zero_hist(hist, nbins)
    if sel_hbm is not None:
        pltpu.sync_copy(sel_hbm.at[pl.ds(0, L)], selv)
        sel = selv[...]
    ones = jnp.ones((L,), jnp.int32)

    def body(i, c):
        xv = xs[pl.ds(i * L, L)]
        bits = _bits_of(xv)
        pos = bits > 0
        if level == 1:
            mask = pos
            binv = bits >> 20
        elif level == 2:
            mask = pos & ((bits >> 20) == sel)
            binv = (bits >> 8) & 0xFFF
        else:
            mask = pos & ((bits >> 8) == sel)
            binv = bits & 0xFF
        plsc.addupdate_scatter(hist, [binv], ones, mask=mask)
        return c

    lax.fori_loop(0, ITERS, body, 0, unroll=4)
    pltpu.sync_copy(hist, out_hbm.at[pl.ds(wid * nbins, nbins)])


def _make_hist(nbins, level):
    if level == 1:
        def body(x_hbm, out_hbm, xs, hist):
            _hist_body(x_hbm, None, out_hbm, xs, hist, None,
                       nbins=nbins, level=level)
    else:
        def body(x_hbm, sel_hbm, out_hbm, xs, hist, selv):
            _hist_body(x_hbm, sel_hbm, out_hbm, xs, hist, selv,
                       nbins=nbins, level=level)
    scratch = [pltpu.VMEM((SHARD,), jnp.float32),
               pltpu.VMEM((nbins,), jnp.int32)]
    if level != 1:
        scratch.append(pltpu.VMEM((L,), jnp.int32))
    return pl.kernel(
        body,
        out_type=jax.ShapeDtypeStruct((NW * nbins,), jnp.int32),
        mesh=_MESH,
        scratch_types=scratch,
        compiler_params=_SC_PARAMS,
        name=f"sc_hist{level}",
    )


_hist1 = _make_hist(4096, 1)
_hist2 = _make_hist(4096, 2)
_hist3 = _make_hist(256, 3)


def _suffix_ge(tot, iota, j):
    return jnp.sum(jnp.where(iota >= j, tot, 0))


def _bin_search(tot, nbits, r):
    iota = lax.broadcasted_iota(jnp.int32, tot.shape, 1)
    b = jnp.int32(0)
    for k in reversed(range(nbits)):
        cand = b + (1 << k)
        b = jnp.where(_suffix_ge(tot, iota, cand) >= r, cand, b)
    return b, _suffix_ge(tot, iota, b + 1)


def _sel1_body(h_ref, out_ref):
    tot = jnp.sum(h_ref[...], axis=0, keepdims=True)
    total = jnp.sum(tot)
    t0 = (total < KK).astype(jnp.int32)
    b1, above = _bin_search(tot, 12, jnp.int32(KK))
    rows = jnp.stack([b1, above, t0, jnp.int32(0)])
    out_ref[...] = jnp.broadcast_to(rows[:, None], (4, L))


def _sel2_body(h_ref, s1_ref, out_ref):
    s1 = s1_ref[...]
    b1, above1 = s1[0, 0], s1[1, 0]
    tot = jnp.sum(h_ref[...], axis=0, keepdims=True)
    b2, sfx = _bin_search(tot, 12, KK - above1)
    p24 = (b1 << 12) | b2
    above2 = above1 + sfx
    rows = jnp.stack([p24, above2, s1[2, 0], jnp.int32(0)])
    out_ref[...] = jnp.broadcast_to(rows[:, None], (4, L))


def _sel3_body(h_ref, s2_ref, out_ref):
    s2 = s2_ref[...]
    p24, above2, t0 = s2[0, 0], s2[1, 0], s2[2, 0]
    h = h_ref[...]
    tot = jnp.sum(h, axis=0, keepdims=True)
    b3, sfx = _bin_search(tot, 8, KK - above2)
    count_greater = above2 + sfx
    t_bits = jnp.where(t0 > 0, 0, (p24 << 8) | b3)
    need = jnp.where(t0 > 0, 0, KK - count_greater)
    cols = lax.broadcasted_iota(jnp.int32, h.shape, 1)
    eq_col = jnp.sum(jnp.where(cols == b3, h, 0), axis=1)
    wi = lax.broadcasted_iota(jnp.int32, (NW, NW), 0)
    wj = lax.broadcasted_iota(jnp.int32, (NW, NW), 1)
    pref = jnp.sum(jnp.where(wj < wi, eq_col[None, :], 0), axis=1)
    head = jnp.broadcast_to(jnp.stack([t_bits, need])[:, None], (2, L))
    body = jnp.broadcast_to(pref[:, None], (NW, L))
    out_ref[...] = jnp.concatenate([head, body], axis=0)


def _final_body(x_hbm, sel_hbm, out_hbm, xs, tv, nv, ev):
    wid = _wid()
    pltpu.sync_copy(x_hbm.at[pl.ds(wid * SHARD, SHARD)], xs)
    pltpu.sync_copy(sel_hbm.at[pl.ds(0, L)], tv)
    pltpu.sync_copy(sel_hbm.at[pl.ds(L, L)], nv)
    pltpu.sync_copy(sel_hbm.at[pl.ds((2 + wid) * L, L)], ev)
    t = tv[...]
    needv = nv[...]
    base = ev[...]

    def body(i, carry):
        xv = xs[pl.ds(i * L, L)]
        bits = _bits_of(xv)
        gt = bits > t
        eq = bits == t
        eqi = jnp.where(eq, 1, 0)
        excl = plsc.cumsum(eqi) - eqi
        rank = base + carry + excl
        keep = gt | (eq & (rank < needv))
        xs[pl.ds(i * L, L)] = jnp.where(keep, jnp.maximum(xv, 0.0), 0.0)
        return carry + plsc.all_reduce_population_count(eq)

    lax.fori_loop(0, ITERS, body, jnp.zeros((L,), jnp.int32), unroll=4)
    pltpu.sync_copy(xs, out_hbm.at[pl.ds(wid * SHARD, SHARD)])


_final = pl.kernel(
    _final_body,
    out_type=jax.ShapeDtypeStruct((N,), jnp.float32),
    mesh=_MESH,
    scratch_types=[pltpu.VMEM((SHARD,), jnp.float32),
                   pltpu.VMEM((L,), jnp.int32),
                   pltpu.VMEM((L,), jnp.int32),
                   pltpu.VMEM((L,), jnp.int32)],
    compiler_params=_SC_PARAMS,
    name="sc_final_mask",
)


def _tc(body, out_shape, name):
    return pl.pallas_call(body, out_shape=out_shape, name=name)


def kernel(x):
    xf = x.reshape(-1)
    h1 = _hist1(xf)
    s1 = _tc(_sel1_body, jax.ShapeDtypeStruct((4, L), jnp.int32),
             "tc_sel1")(h1.reshape(NW, 4096))
    h2 = _hist2(xf, s1.reshape(-1))
    s2 = _tc(_sel2_body, jax.ShapeDtypeStruct((4, L), jnp.int32),
             "tc_sel2")(h2.reshape(NW, 4096), s1)
    h3 = _hist3(xf, s2.reshape(-1))
    s3 = _tc(_sel3_body, jax.ShapeDtypeStruct((2 + NW, L), jnp.int32),
             "tc_sel3")(h3.reshape(NW, 256), s2)
    out = _final(xf, s3.reshape(-1))
    return out.reshape(x.shape)


# speedup vs baseline: 8.6167x; 1.0531x over previous
import jax
import jax.numpy as jnp
from jax import lax
from jax.experimental import pallas as pl
from jax.experimental.pallas import tpu as pltpu
from jax.experimental.pallas import tpu_sc as plsc

NC = 2
NS = 16
L = 16
NW = NC * NS
N = 128 * 8192
SHARD = N // NW
ITERS = SHARD // L
KK = 8192
NB12 = 4096
NB3 = 256

_MESH = plsc.VectorSubcoreMesh(
    core_axis_name="c", subcore_axis_name="s", num_cores=NC, num_subcores=NS)
_SC_PARAMS = pltpu.CompilerParams(needs_layout_passes=False)


def _bits_vec(xv):
    return jnp.maximum(plsc.bitcast(xv, jnp.int32), 0)


def _body(x_hbm, out_hbm, xchg_hbm, h3x_hbm, xs, hist, tmp, h3all, shared,
          shared_red, gpref, sem):
    cid = lax.axis_index("c")
    sid = lax.axis_index("s")
    wid = sid * NC + cid
    pltpu.sync_copy(x_hbm.at[pl.ds(wid * SHARD, SHARD)], xs)

    def hist_pass(nbins, mode, selp):
        z = jnp.zeros((L,), jnp.int32)

        def zb(i, c):
            hist[pl.ds(i * L, L)] = z
            return c

        lax.fori_loop(0, nbins // L, zb, 0, unroll=4)
        ones = jnp.ones((L,), jnp.int32)
        selv = None if selp is None else jnp.full((L,), selp, jnp.int32)

        def hb(i, c):
            xv = xs[pl.ds(i * L, L)]
            bits = _bits_vec(xv)
            if mode == 1:
                plsc.addupdate_scatter(hist, [bits >> 20], ones)
            elif mode == 2:
                mask = (bits >> 20) == selv
                plsc.addupdate_scatter(hist, [(bits >> 8) & 0xFFF], ones,
                                       mask=mask)
            else:
                mask = (bits >> 8) == selv
                plsc.addupdate_scatter(hist, [bits & 0xFF], ones, mask=mask)
            return c

        lax.fori_loop(0, ITERS, hb, 0, unroll=4)

    def reduce_exchange(nbins):
        nred = nbins // NS
        pltpu.sync_copy(hist.at[pl.ds(0, nbins)],
                        shared.at[pl.ds(sid * NB12, nbins)])
        plsc.subcore_barrier()
        for r in range(NS):
            pltpu.sync_copy(shared.at[pl.ds(r * NB12 + sid * nred, nred)],
                            tmp.at[pl.ds(r * nred, nred)])
        for g in range(nred // L):
            def rb(r, acc):
                return acc + tmp[pl.ds(r * nred + g * L, L)]

            hist[pl.ds(g * L, L)] = lax.fori_loop(
                0, NS, rb, jnp.zeros((L,), jnp.int32), unroll=4)
        pltpu.sync_copy(hist.at[pl.ds(0, nred)],
                        shared_red.at[pl.ds(sid * nred, nred)])
        plsc.subcore_barrier()

        @pl.when(sid == 0)
        def _():
            pltpu.sync_copy(shared_red.at[pl.ds(0, nbins)],
                            xchg_hbm.at[pl.ds(cid * NB12, nbins)])

        pltpu.core_barrier(sem, core_axis_name="c")
        plsc.subcore_barrier()
        pltpu.sync_copy(shared_red.at[pl.ds(0, nbins)], tmp.at[pl.ds(0, nbins)])
        pltpu.sync_copy(xchg_hbm.at[pl.ds((1 - cid) * NB12, nbins)],
                        hist.at[pl.ds(0, nbins)])

        def cb(i, c):
            v = tmp[pl.ds(i * L, L)] + hist[pl.ds(i * L, L)]
            tmp[pl.ds(i * L, L)] = plsc.cumsum(v)
            return c

        lax.fori_loop(0, nbins // L, cb, 0, unroll=4)

    def group_prefix(nbins):
        ng = nbins // L

        def gb(g, acc):
            gpref[g] = acc
            return acc + tmp[pl.ds(g * L, L)][L - 1]

        return lax.fori_loop(0, ng, gb, jnp.int32(0))

    def make_count_ge(nbins, tot):
        ng = nbins // L

        def count_ge(b):
            g = jnp.minimum(b >> 4, ng - 1)
            r = b & (L - 1)
            tprev = tmp[pl.ds(jnp.maximum(b - 1, 0), L)][0]
            pe = gpref[g] + jnp.where(r > 0, tprev, 0)
            return jnp.where(b >= nbins, 0, tot - pe)

        return count_ge

    def bsearch(count_ge, nbits, r):
        def sb(k, b):
            cand = b + lax.shift_left(jnp.int32(1), nbits - 1 - k)
            return jnp.where(count_ge(cand) >= r, cand, b)

        b = lax.fori_loop(0, nbits, sb, jnp.int32(0))
        return b, count_ge(b + 1)

    hist_pass(NB12, 1, None)
    reduce_exchange(NB12)
    tot1 = group_prefix(NB12)
    cg1 = make_count_ge(NB12, tot1)
    b1, above1 = bsearch(cg1, 12, jnp.int32(KK))

    hist_pass(NB12, 2, b1)
    reduce_exchange(NB12)
    tot2 = group_prefix(NB12)
    cg2 = make_count_ge(NB12, tot2)
    b2, sfx2 = bsearch(cg2, 12, KK - above1)
    p24 = lax.shift_left(b1, 12) | b2
    above2 = above1 + sfx2

    hist_pass(NB3, 3, p24)
    pltpu.sync_copy(hist.at[pl.ds(0, NB3)], h3x_hbm.at[pl.ds(wid * NB3, NB3)])
    pltpu.core_barrier(sem, core_axis_name="c")
    plsc.subcore_barrier()
    pltpu.sync_copy(h3x_hbm, h3all.at[pl.ds(0, NW * NB3)])
    for g in range(NB3 // L):
        def ab(w, acc):
            return acc + h3all[pl.ds(w * NB3 + g * L, L)]

        v = lax.fori_loop(0, NW, ab, jnp.zeros((L,), jnp.int32), unroll=4)
        tmp[pl.ds(g * L, L)] = plsc.cumsum(v)
    tot3 = group_prefix(NB3)
    cg3 = make_count_ge(NB3, tot3)
    b3, sfx3 = bsearch(cg3, 8, KK - above2)
    count_greater = above2 + sfx3
    t_bits = lax.shift_left(p24, 8) | b3
    need = KK - count_greater

    def pb(w, acc):
        cnt = h3all[pl.ds(w * NB3 + b3, L)][0]
        return acc + jnp.where(w < wid, cnt, 0)

    base = lax.fori_loop(0, NW, pb, jnp.int32(0))
    my_eq = h3all[pl.ds(wid * NB3 + b3, L)][0]

    tv = jnp.full((L,), t_bits, jnp.int32)

    @pl.when(my_eq == 0)
    def _():
        def fb(i, c):
            xv = xs[pl.ds(i * L, L)]
            keep = _bits_vec(xv) > tv
            xs[pl.ds(i * L, L)] = jnp.where(keep, xv, 0.0)
            return c

        lax.fori_loop(0, ITERS, fb, 0, unroll=8)

    @pl.when(my_eq != 0)
    def _():
        needv = jnp.full((L,), need, jnp.int32)
        basev = jnp.full((L,), base, jnp.int32)

        def fb(i, carry):
            xv = xs[pl.ds(i * L, L)]
            bits = _bits_vec(xv)
            gt = bits > tv
            eq = bits == tv
            eqi = jnp.where(eq, 1, 0)
            excl = plsc.cumsum(eqi) - eqi
            rank = basev + carry + excl
            keep = gt | (eq & (rank < needv))
            xs[pl.ds(i * L, L)] = jnp.where(keep, jnp.maximum(xv, 0.0), 0.0)
            return carry + plsc.all_reduce_population_count(eq)

        lax.fori_loop(0, ITERS, fb, jnp.zeros((L,), jnp.int32), unroll=4)

    pltpu.sync_copy(xs, out_hbm.at[pl.ds(wid * SHARD, SHARD)])


_fused = pl.kernel(
    _body,
    out_type=(jax.ShapeDtypeStruct((N,), jnp.float32),
              jax.ShapeDtypeStruct((NC * NB12,), jnp.int32),
              jax.ShapeDtypeStruct((NW * NB3,), jnp.int32)),
    mesh=_MESH,
    scratch_types=[pltpu.VMEM((SHARD,), jnp.float32),
                   pltpu.VMEM((NB12,), jnp.int32),
                   pltpu.VMEM((NB12 + L,), jnp.int32),
                   pltpu.VMEM((NW * NB3 + L,), jnp.int32),
                   pltpu.VMEM_SHARED((NS * NB12,), jnp.int32),
                   pltpu.VMEM_SHARED((NB12,), jnp.int32),
                   pltpu.SMEM((NB12 // L,), jnp.int32),
                   pltpu.SemaphoreType.REGULAR],
    compiler_params=_SC_PARAMS,
    name="sc_topk_fused",
)


def kernel(x):
    out, _, _ = _fused(x.reshape(-1))
    return out.reshape(x.shape)


# speedup vs baseline: 10.5087x; 1.2196x over previous
import jax
import jax.numpy as jnp
from jax import lax
from jax.experimental import pallas as pl
from jax.experimental.pallas import tpu as pltpu
from jax.experimental.pallas import tpu_sc as plsc

NC = 2
NS = 16
L = 16
NW = NC * NS
N = 128 * 8192
SHARD = N // NW
ITERS = SHARD // L
KK = 8192
NB12 = 4096
NB3 = 256

_MESH = plsc.VectorSubcoreMesh(
    core_axis_name="c", subcore_axis_name="s", num_cores=NC, num_subcores=NS)
_SC_PARAMS = pltpu.CompilerParams(needs_layout_passes=False)


def _bits_vec(xv):
    return jnp.maximum(plsc.bitcast(xv, jnp.int32), 0)


def _body(x_hbm, out_hbm, xchg_hbm, h3x_hbm, xs, hist, tmp, h3all, keys, idxs,
          shared, shared_red, gpref, sem):
    cid = lax.axis_index("c")
    sid = lax.axis_index("s")
    wid = sid * NC + cid
    pltpu.sync_copy(x_hbm.at[pl.ds(wid * SHARD, SHARD)], xs)
    ones = jnp.ones((L,), jnp.int32)

    def zero_hist(nbins):
        z = jnp.zeros((L,), jnp.int32)

        def zb(i, c):
            hist[pl.ds(i * L, L)] = z
            return c

        lax.fori_loop(0, nbins // L, zb, 0, unroll=4)

    def hist1_pass():
        zero_hist(NB12)

        def hb(i, c):
            xv = xs[pl.ds(i * L, L)]
            bits = _bits_vec(xv)
            plsc.addupdate_scatter(hist, [bits >> 20], ones)
            return c

        lax.fori_loop(0, ITERS, hb, 0, unroll=4)

    def compact_hist_pass(nbins, shift, mask_shift, selp, mi):
        zero_hist(nbins)
        selv = jnp.full((L,), selp, jnp.int32)

        def hb(i, c):
            kv = keys[pl.ds(i * L, L)]
            mask = (kv >> mask_shift) == selv
            plsc.addupdate_scatter(hist, [(kv >> shift) & (nbins - 1)], ones,
                                   mask=mask)
            return c

        lax.fori_loop(0, mi, hb, 0)

    def reduce_exchange(nbins):
        nred = nbins // NS
        pltpu.sync_copy(hist.at[pl.ds(0, nbins)],
                        shared.at[pl.ds(sid * NB12, nbins)])
        plsc.subcore_barrier()
        for r in range(NS):
            pltpu.sync_copy(shared.at[pl.ds(r * NB12 + sid * nred, nred)],
                            tmp.at[pl.ds(r * nred, nred)])
        for g in range(nred // L):
            def rb(r, acc):
                return acc + tmp[pl.ds(r * nred + g * L, L)]

            hist[pl.ds(g * L, L)] = lax.fori_loop(
                0, NS, rb, jnp.zeros((L,), jnp.int32), unroll=4)
        pltpu.sync_copy(hist.at[pl.ds(0, nred)],
                        shared_red.at[pl.ds(sid * nred, nred)])
        plsc.subcore_barrier()

        @pl.when(sid == 0)
        def _():
            pltpu.sync_copy(shared_red.at[pl.ds(0, nbins)],
                            xchg_hbm.at[pl.ds(cid * NB12, nbins)])

        pltpu.core_barrier(sem, core_axis_name="c")
        plsc.subcore_barrier()
        pltpu.sync_copy(shared_red.at[pl.ds(0, nbins)], tmp.at[pl.ds(0, nbins)])
        pltpu.sync_copy(xchg_hbm.at[pl.ds((1 - cid) * NB12, nbins)],
                        hist.at[pl.ds(0, nbins)])

        def cb(i, c):
            v = tmp[pl.ds(i * L, L)] + hist[pl.ds(i * L, L)]
            tmp[pl.ds(i * L, L)] = plsc.cumsum(v)
            return c

        lax.fori_loop(0, nbins // L, cb, 0, unroll=4)

    def group_prefix(nbins):
        ng = nbins // L

        def gb(g, acc):
            gpref[g] = acc
            return acc + tmp[pl.ds(g * L, L)][L - 1]

        return lax.fori_loop(0, ng, gb, jnp.int32(0))

    def make_count_ge(nbins, tot):
        ng = nbins // L

        def count_ge(b):
            g = jnp.minimum(b >> 4, ng - 1)
            r = b & (L - 1)
            tprev = tmp[pl.ds(jnp.maximum(b - 1, 0), L)][0]
            pe = gpref[g] + jnp.where(r > 0, tprev, 0)
            return jnp.where(b >= nbins, 0, tot - pe)

        return count_ge

    def bsearch(count_ge, nbits, r):
        def sb(k, b):
            cand = b + lax.shift_left(jnp.int32(1), nbits - 1 - k)
            return jnp.where(count_ge(cand) >= r, cand, b)

        b = lax.fori_loop(0, nbits, sb, jnp.int32(0))
        return b, count_ge(b + 1)

    hist1_pass()
    reduce_exchange(NB12)
    tot1 = group_prefix(NB12)
    cg1 = make_count_ge(NB12, tot1)
    b1, above1 = bsearch(cg1, 12, jnp.int32(KK))

    b1v = jnp.full((L,), b1, jnp.int32)
    iota0 = lax.broadcasted_iota(jnp.int32, (L,), 0)

    def cp(i, cnt):
        xv = xs[pl.ds(i * L, L)]
        key = _bits_vec(xv)
        mge = (key >> 20) >= b1v
        c0 = cnt[0]
        plsc.store_compressed(keys.at[pl.ds(c0, L)], key, mask=mge)
        plsc.store_compressed(idxs.at[pl.ds(c0, L)], iota0 + i * L, mask=mge)
        return cnt + plsc.all_reduce_population_count(mge)

    cntv = lax.fori_loop(0, ITERS, cp, jnp.zeros((L,), jnp.int32), unroll=4)
    m = cntv[0]
    keys[pl.ds(m, L)] = jnp.full((L,), -1, jnp.int32)
    mi = (m + L - 1) // L

    zf = jnp.zeros((L,), jnp.float32)

    def zx(i, c):
        xs[pl.ds(i * L, L)] = zf
        return c

    lax.fori_loop(0, ITERS, zx, 0, unroll=8)

    compact_hist_pass(NB12, 8, 20, b1, mi)
    reduce_exchange(NB12)
    tot2 = group_prefix(NB12)
    cg2 = make_count_ge(NB12, tot2)
    b2, sfx2 = bsearch(cg2, 12, KK - above1)
    p24 = lax.shift_left(b1, 12) | b2
    above2 = above1 + sfx2

    compact_hist_pass(NB3, 0, 8, p24, mi)
    pltpu.sync_copy(hist.at[pl.ds(0, NB3)], h3x_hbm.at[pl.ds(wid * NB3, NB3)])
    pltpu.core_barrier(sem, core_axis_name="c")
    plsc.subcore_barrier()
    pltpu.sync_copy(h3x_hbm, h3all.at[pl.ds(0, NW * NB3)])
    for g in range(NB3 // L):
        def ab(w, acc):
            return acc + h3all[pl.ds(w * NB3 + g * L, L)]

        v = lax.fori_loop(0, NW, ab, jnp.zeros((L,), jnp.int32), unroll=4)
        tmp[pl.ds(g * L, L)] = plsc.cumsum(v)
    tot3 = group_prefix(NB3)
    cg3 = make_count_ge(NB3, tot3)
    b3, sfx3 = bsearch(cg3, 8, KK - above2)
    count_greater = above2 + sfx3
    t_bits = lax.shift_left(p24, 8) | b3
    need = KK - count_greater

    def pb(w, acc):
        cnt = h3all[pl.ds(w * NB3 + b3, L)][0]
        return acc + jnp.where(w < wid, cnt, 0)

    base = lax.fori_loop(0, NW, pb, jnp.int32(0))

    tv = jnp.full((L,), t_bits, jnp.int32)
    needv = jnp.full((L,), need, jnp.int32)
    basev = jnp.full((L,), base, jnp.int32)

    def ws(i, carry):
        kv = keys[pl.ds(i * L, L)]
        iv = idxs[pl.ds(i * L, L)]
        gt = kv > tv
        eq = kv == tv
        eqi = jnp.where(eq, 1, 0)
        excl = plsc.cumsum(eqi) - eqi
        rank = basev + carry + excl
        keep = gt | (eq & (rank < needv))
        plsc.store_scatter(xs, [iv], plsc.bitcast(kv, jnp.float32), mask=keep)
        return carry + plsc.all_reduce_population_count(eq)

    lax.fori_loop(0, mi, ws, jnp.zeros((L,), jnp.int32))

    pltpu.sync_copy(xs, out_hbm.at[pl.ds(wid * SHARD, SHARD)])


_fused = pl.kernel(
    _body,
    out_type=(jax.ShapeDtypeStruct((N,), jnp.float32),
              jax.ShapeDtypeStruct((NC * NB12,), jnp.int32),
              jax.ShapeDtypeStruct((NW * NB3,), jnp.int32)),
    mesh=_MESH,
    scratch_types=[pltpu.VMEM((SHARD,), jnp.float32),
                   pltpu.VMEM((NB12,), jnp.int32),
                   pltpu.VMEM((NB12 + L,), jnp.int32),
                   pltpu.VMEM((NW * NB3 + L,), jnp.int32),
                   pltpu.VMEM((SHARD + L,), jnp.int32),
                   pltpu.VMEM((SHARD + L,), jnp.int32),
                   pltpu.VMEM_SHARED((NS * NB12,), jnp.int32),
                   pltpu.VMEM_SHARED((NB12,), jnp.int32),
                   pltpu.SMEM((NB12 // L,), jnp.int32),
                   pltpu.SemaphoreType.REGULAR],
    compiler_params=_SC_PARAMS,
    name="sc_topk_fused",
)


def kernel(x):
    out, _, _ = _fused(x.reshape(-1))
    return out.reshape(x.shape)


# speedup vs baseline: 11.1397x; 1.0600x over previous
import jax
import jax.numpy as jnp
from jax import lax
from jax.experimental import pallas as pl
from jax.experimental.pallas import tpu as pltpu
from jax.experimental.pallas import tpu_sc as plsc

NC = 2
NS = 16
L = 16
NW = NC * NS
N = 128 * 8192
SHARD = N // NW
ITERS = SHARD // L
KK = 8192
NB12 = 4096
NB3 = 256

_MESH = plsc.VectorSubcoreMesh(
    core_axis_name="c", subcore_axis_name="s", num_cores=NC, num_subcores=NS)
_SC_PARAMS = pltpu.CompilerParams(needs_layout_passes=False)


def _bits_vec(xv):
    return jnp.maximum(plsc.bitcast(xv, jnp.int32), 0)


def _body(x_hbm, out_hbm, xchg_hbm, h3x_hbm, xs, hist, tmp, h3all, keys, idxs,
          shared, shared_red, gpref, sem):
    cid = lax.axis_index("c")
    sid = lax.axis_index("s")
    wid = sid * NC + cid
    pltpu.sync_copy(x_hbm.at[pl.ds(wid * SHARD, SHARD)], xs)
    ones = jnp.ones((L,), jnp.int32)

    def zero_hist(nbins):
        z = jnp.zeros((L,), jnp.int32)

        def zb(i, c):
            hist[pl.ds(i * L, L)] = z
            return c

        lax.fori_loop(0, nbins // L, zb, 0, unroll=4)

    def hist1_pass():
        zero_hist(NB12)

        zv = jnp.zeros((L,), jnp.int32)

        def hb(i, c):
            xv = xs[pl.ds(i * L, L)]
            bits = _bits_vec(xv)
            plsc.addupdate_scatter(hist, [bits >> 20], ones, mask=bits > zv)
            return c

        lax.fori_loop(0, ITERS, hb, 0, unroll=8)

    def compact_hist_pass(nbins, shift, mask_shift, selp, mi):
        zero_hist(nbins)
        selv = jnp.full((L,), selp, jnp.int32)

        def hb(i, c):
            kv = keys[pl.ds(i * L, L)]
            mask = (kv >> mask_shift) == selv
            plsc.addupdate_scatter(hist, [(kv >> shift) & (nbins - 1)], ones,
                                   mask=mask)
            return c

        lax.fori_loop(0, mi, hb, 0)

    def reduce_exchange(nbins):
        nred = nbins // NS
        pltpu.sync_copy(hist.at[pl.ds(0, nbins)],
                        shared.at[pl.ds(sid * NB12, nbins)])
        plsc.subcore_barrier()
        for r in range(NS):
            pltpu.sync_copy(shared.at[pl.ds(r * NB12 + sid * nred, nred)],
                            tmp.at[pl.ds(r * nred, nred)])
        for g in range(nred // L):
            def rb(r, acc):
                return acc + tmp[pl.ds(r * nred + g * L, L)]

            hist[pl.ds(g * L, L)] = lax.fori_loop(
                0, NS, rb, jnp.zeros((L,), jnp.int32), unroll=4)
        pltpu.sync_copy(hist.at[pl.ds(0, nred)],
                        shared_red.at[pl.ds(sid * nred, nred)])
        plsc.subcore_barrier()

        @pl.when(sid == 0)
        def _():
            pltpu.sync_copy(shared_red.at[pl.ds(0, nbins)],
                            xchg_hbm.at[pl.ds(cid * NB12, nbins)])

        pltpu.core_barrier(sem, core_axis_name="c")
        plsc.subcore_barrier()
        pltpu.sync_copy(shared_red.at[pl.ds(0, nbins)], tmp.at[pl.ds(0, nbins)])
        pltpu.sync_copy(xchg_hbm.at[pl.ds((1 - cid) * NB12, nbins)],
                        hist.at[pl.ds(0, nbins)])

        def cb(i, c):
            v = tmp[pl.ds(i * L, L)] + hist[pl.ds(i * L, L)]
            tmp[pl.ds(i * L, L)] = plsc.cumsum(v)
            return c

        lax.fori_loop(0, nbins // L, cb, 0, unroll=4)

    def group_prefix(nbins):
        ng = nbins // L

        def gb(g, acc):
            gpref[g] = acc
            return acc + tmp[pl.ds(g * L, L)][L - 1]

        return lax.fori_loop(0, ng, gb, jnp.int32(0))

    def make_count_ge(nbins, tot):
        ng = nbins // L

        def count_ge(b):
            g = jnp.minimum(b >> 4, ng - 1)
            r = b & (L - 1)
            tprev = tmp[pl.ds(jnp.maximum(b - 1, 0), L)][0]
            pe = gpref[g] + jnp.where(r > 0, tprev, 0)
            return jnp.where(b >= nbins, 0, tot - pe)

        return count_ge

    def bsearch(count_ge, nbits, r):
        def sb(k, b):
            cand = b + lax.shift_left(jnp.int32(1), nbits - 1 - k)
            return jnp.where(count_ge(cand) >= r, cand, b)

        b = lax.fori_loop(0, nbits, sb, jnp.int32(0))
        return b, count_ge(b + 1)

    hist1_pass()
    reduce_exchange(NB12)
    tot1 = group_prefix(NB12)
    cg1 = make_count_ge(NB12, tot1)
    b1, above1 = bsearch(cg1, 12, jnp.int32(KK))

    b1v = jnp.full((L,), b1, jnp.int32)
    iota0 = lax.broadcasted_iota(jnp.int32, (L,), 0)

    def cp(i, cnt):
        xv = xs[pl.ds(i * L, L)]
        key = _bits_vec(xv)
        mge = (key >> 20) >= b1v
        c0 = cnt[0]
        plsc.store_compressed(keys.at[pl.ds(c0, L)], key, mask=mge)
        plsc.store_compressed(idxs.at[pl.ds(c0, L)], iota0 + i * L, mask=mge)
        return cnt + plsc.all_reduce_population_count(mge)

    cntv = lax.fori_loop(0, ITERS, cp, jnp.zeros((L,), jnp.int32), unroll=8)
    m = cntv[0]
    keys[pl.ds(m, L)] = jnp.full((L,), -1, jnp.int32)
    mi = (m + L - 1) // L

    zf = jnp.zeros((L,), jnp.float32)

    def zx(i, c):
        xs[pl.ds(i * L, L)] = zf
        return c

    lax.fori_loop(0, ITERS, zx, 0, unroll=8)

    compact_hist_pass(NB12, 8, 20, b1, mi)
    reduce_exchange(NB12)
    tot2 = group_prefix(NB12)
    cg2 = make_count_ge(NB12, tot2)
    b2, sfx2 = bsearch(cg2, 12, KK - above1)
    p24 = lax.shift_left(b1, 12) | b2
    above2 = above1 + sfx2

    compact_hist_pass(NB3, 0, 8, p24, mi)
    pltpu.sync_copy(hist.at[pl.ds(0, NB3)], h3x_hbm.at[pl.ds(wid * NB3, NB3)])
    pltpu.core_barrier(sem, core_axis_name="c")
    plsc.subcore_barrier()
    pltpu.sync_copy(h3x_hbm, h3all.at[pl.ds(0, NW * NB3)])
    for g in range(NB3 // L):
        def ab(w, acc):
            return acc + h3all[pl.ds(w * NB3 + g * L, L)]

        v = lax.fori_loop(0, NW, ab, jnp.zeros((L,), jnp.int32), unroll=4)
        tmp[pl.ds(g * L, L)] = plsc.cumsum(v)
    tot3 = group_prefix(NB3)
    cg3 = make_count_ge(NB3, tot3)
    b3, sfx3 = bsearch(cg3, 8, KK - above2)
    count_greater = above2 + sfx3
    t_bits = lax.shift_left(p24, 8) | b3
    need = KK - count_greater

    def pb(w, acc):
        cnt = h3all[pl.ds(w * NB3 + b3, L)][0]
        return acc + jnp.where(w < wid, cnt, 0)

    base = lax.fori_loop(0, NW, pb, jnp.int32(0))

    tv = jnp.full((L,), t_bits, jnp.int32)
    needv = jnp.full((L,), need, jnp.int32)
    basev = jnp.full((L,), base, jnp.int32)

    def ws(i, carry):
        kv = keys[pl.ds(i * L, L)]
        iv = idxs[pl.ds(i * L, L)]
        gt = kv > tv
        eq = kv == tv
        eqi = jnp.where(eq, 1, 0)
        excl = plsc.cumsum(eqi) - eqi
        rank = basev + carry + excl
        keep = gt | (eq & (rank < needv))
        plsc.store_scatter(xs, [iv], plsc.bitcast(kv, jnp.float32), mask=keep)
        return carry + plsc.all_reduce_population_count(eq)

    lax.fori_loop(0, mi, ws, jnp.zeros((L,), jnp.int32))

    pltpu.sync_copy(xs, out_hbm.at[pl.ds(wid * SHARD, SHARD)])


_fused = pl.kernel(
    _body,
    out_type=(jax.ShapeDtypeStruct((N,), jnp.float32),
              jax.ShapeDtypeStruct((NC * NB12,), jnp.int32),
              jax.ShapeDtypeStruct((NW * NB3,), jnp.int32)),
    mesh=_MESH,
    scratch_types=[pltpu.VMEM((SHARD,), jnp.float32),
                   pltpu.VMEM((NB12,), jnp.int32),
                   pltpu.VMEM((NB12 + L,), jnp.int32),
                   pltpu.VMEM((NW * NB3 + L,), jnp.int32),
                   pltpu.VMEM((SHARD + L,), jnp.int32),
                   pltpu.VMEM((SHARD + L,), jnp.int32),
                   pltpu.VMEM_SHARED((NS * NB12,), jnp.int32),
                   pltpu.VMEM_SHARED((NB12,), jnp.int32),
                   pltpu.SMEM((NB12 // L,), jnp.int32),
                   pltpu.SemaphoreType.REGULAR],
    compiler_params=_SC_PARAMS,
    name="sc_topk_fused",
)


def kernel(x):
    out, _, _ = _fused(x.reshape(-1))
    return out.reshape(x.shape)
